# baseline (speedup 1.0000x reference)
import functools

import jax
import jax.numpy as jnp
from jax import lax
from jax.experimental import pallas as pl
from jax.experimental.pallas import tpu as pltpu
from jax.experimental.pallas import tpu_sc as plsc

B, N, M = 8, 2048, 2048
L = 16
TR = 16
R = B * N
NT = R // TR
NW = 32
TPW = NT // NW
TILES_PER_BATCH = N // TR
VPR = M // L
UNROLL = 8


def _rsqrt16(ssv):
    iv = lax.bitcast_convert_type(ssv, jnp.int32)
    iv = jnp.int32(0x5F3759DF) - (iv >> 1)
    y = lax.bitcast_convert_type(iv, jnp.float32)
    for _ in range(3):
        y = y * (1.5 - 0.5 * ssv * y * y)
    return y


def _row_softmax(buf, r):
    def ssq_step(j, acc):
        base = j * (L * UNROLL)
        for u in range(UNROLL):
            v = buf[r, pl.ds(base + u * L, L)]
            acc = acc + v * v
        return acc

    acc = lax.fori_loop(0, VPR // UNROLL, ssq_step, jnp.zeros((L,), jnp.float32))
    ss = jnp.sum(acc)
    ssv = jnp.full((L,), ss)
    rv = jnp.minimum(_rsqrt16(ssv), jnp.float32(1e12))

    def exp_step(j, sacc):
        base = j * (L * UNROLL)
        for u in range(UNROLL):
            v = buf[r, pl.ds(base + u * L, L)]
            e = jnp.exp(v * rv)
            buf[r, pl.ds(base + u * L, L)] = e
            sacc = sacc + e
        return sacc

    sacc = lax.fori_loop(0, VPR // UNROLL, exp_step,
                         jnp.zeros((L,), jnp.float32))
    inv = jnp.ones((L,), jnp.float32) / jnp.full((L,), jnp.sum(sacc))

    def scale_step(j, c):
        base = j * (L * UNROLL)
        for u in range(UNROLL):
            buf[r, pl.ds(base + u * L, L)] = buf[r, pl.ds(base + u * L, L)] * inv
        return c

    lax.fori_loop(0, VPR // UNROLL, scale_step, jnp.int32(0))


def _zero_rows(buf, lo, hi):
    z = jnp.zeros((L,), jnp.float32)

    def row_step(r, c):
        def col_step(j, c2):
            base = j * (L * UNROLL)
            for u in range(UNROLL):
                buf[r, pl.ds(base + u * L, L)] = z
            return c2
        return lax.fori_loop(0, VPR // UNROLL, col_step, c)

    lax.fori_loop(lo, hi, row_step, jnp.int32(0))


def _sc_body(s_hbm, nrow_hbm, out_hbm, nrow_v, buf, zbuf):
    wid = lax.axis_index("s") * 2 + lax.axis_index("c")

    pltpu.sync_copy(nrow_hbm, nrow_v)
    nrowf = nrow_v[...].astype(jnp.float32)
    lanes = jnp.arange(L, dtype=jnp.int32)
    _zero_rows(zbuf, 0, TR)

    def tile_step(i, c):
        t = wid + NW * i
        b = t // TILES_PER_BATCH
        start = (t - b * TILES_PER_BATCH) * TR
        nrow_b = jnp.sum(jnp.where(lanes == b, nrowf, 0.0)).astype(jnp.int32)
        nact = jnp.clip(nrow_b - start, 0, TR)

        @pl.when(nact == 0)
        def _():
            pltpu.sync_copy(zbuf, out_hbm.at[pl.ds(t * TR, TR)])

        @pl.when(nact > 0)
        def _():
            pltpu.sync_copy(s_hbm.at[pl.ds(t * TR, TR)], buf)

            def row_step(r, c2):
                _row_softmax(buf, r)
                return c2

            lax.fori_loop(0, nact, row_step, jnp.int32(0))
            _zero_rows(buf, nact, TR)
            pltpu.sync_copy(buf, out_hbm.at[pl.ds(t * TR, TR)])

        return c

    lax.fori_loop(0, TPW, tile_step, jnp.int32(0))


def kernel(s, nrow_gt):
    nrow16 = jnp.zeros((L,), jnp.int32).at[:B].set(nrow_gt.astype(jnp.int32))
    s2 = s.reshape(R, M)
    mesh = plsc.VectorSubcoreMesh(core_axis_name="c", subcore_axis_name="s")
    out = pl.kernel(
        _sc_body,
        mesh=mesh,
        compiler_params=pltpu.CompilerParams(needs_layout_passes=False),
        out_type=jax.ShapeDtypeStruct((R, M), jnp.float32),
        scratch_types=[
            pltpu.VMEM((L,), jnp.int32),
            pltpu.VMEM((TR, M), jnp.float32),
            pltpu.VMEM((TR, M), jnp.float32),
        ],
    )(s2, nrow16)
    return out.reshape(B, N, M)


# speedup vs baseline: 5.7386x; 5.7386x over previous
import functools

import jax
import jax.numpy as jnp
from jax.experimental import pallas as pl
from jax.experimental.pallas import tpu as pltpu

B, N, M = 8, 2048, 2048
BR = 256


def _body(nrow_ref, s_ref, o_ref):
    j = pl.program_id(1)
    nrow = nrow_ref[pl.program_id(0)]
    start = j * BR

    @pl.when(start >= nrow)
    def _zero():
        o_ref[...] = jnp.zeros_like(o_ref)

    @pl.when(start < nrow)
    def _compute():
        x = s_ref[0]
        ss = jnp.sum(x * x, axis=-1, keepdims=True)
        r = 1.0 / jnp.maximum(jnp.sqrt(ss), 1e-12)
        e = jnp.exp(x * r)
        se = jnp.sum(e, axis=-1, keepdims=True)
        out = e / se
        @pl.when(start + BR > nrow)
        def _mask():
            rows = jax.lax.broadcasted_iota(jnp.int32, (BR, M), 0) + start
            o_ref[0] = jnp.where(rows < nrow, out, 0.0)

        @pl.when(start + BR <= nrow)
        def _full():
            o_ref[0] = out


def _s_index(b, j, nrow_ref):
    nrow = nrow_ref[b]
    last_active = jnp.maximum((nrow + BR - 1) // BR - 1, 0)
    return b, jnp.minimum(j, last_active), 0


def _o_index(b, j, nrow_ref):
    return b, j, 0


@functools.partial(jax.jit, static_argnames=())
def kernel(s, nrow_gt):
    nrow = nrow_gt.astype(jnp.int32)
    grid_spec = pltpu.PrefetchScalarGridSpec(
        num_scalar_prefetch=1,
        grid=(B, N // BR),
        in_specs=[pl.BlockSpec((1, BR, M), _s_index)],
        out_specs=pl.BlockSpec((1, BR, M), _o_index),
    )
    return pl.pallas_call(
        _body,
        grid_spec=grid_spec,
        out_shape=jax.ShapeDtypeStruct((B, N, M), jnp.float32),
    )(nrow, s)
